# baseline (speedup 1.0000x reference)
import functools

import jax
import jax.numpy as jnp
from jax import lax
from jax.experimental import pallas as pl
from jax.experimental.pallas import tpu as pltpu
from jax.experimental.pallas import tpu_sc as plsc

N = 10000
NPAD = 10240
E = 320000
D = 128
HD = D // 2
DELTA = 5.27 / 2.0

NC = 2
NS = 16
NW = NC * NS
HCHUNK = 80
HEPT = E // NW
HNCHUNK = HEPT // HCHUNK

CHUNK = 128
NCHUNK = 160
EPT = NCHUNK * CHUNK
EPAD = EPT * NS
NBUF = 4
ROWS_PT = NPAD // NS
HR = NPAD // D


def _mesh():
    return plsc.VectorSubcoreMesh(
        core_axis_name="c", subcore_axis_name="s", num_cores=NC, num_subcores=NS)


def _hist_body(dst_hbm, hist_hbm, dst_v, hist_v, idx_v, hist_sh):
    c = lax.axis_index("c")
    s = lax.axis_index("s")
    wid = c * NS + s
    pltpu.sync_copy(dst_hbm.at[wid], dst_v)

    zeros16 = jnp.zeros((16,), jnp.float32)

    def zrow(r, carry):
        for i in range(D // 16):
            hist_v[r, pl.ds(i * 16, 16)] = zeros16
        return carry

    lax.fori_loop(0, HR, zrow, 0)
    for i in range(HR // 16):
        idx_v[0, pl.ds(i * 16, 16)] = lax.iota(jnp.int32, 16) + (i * 16)

    @pl.when(s == 0)
    def _():
        pltpu.sync_copy(hist_v, hist_sh)

    plsc.subcore_barrier()

    ones16 = jnp.ones((16,), jnp.float32)

    def hrow(r, carry):
        for i in range(HCHUNK // 16):
            idx = dst_v[r, pl.ds(i * 16, 16)]
            row = lax.shift_right_logical(idx, 7)
            col = lax.bitwise_and(idx, 127)
            plsc.addupdate_scatter(hist_v, [row, col], ones16)
        return carry

    lax.fori_loop(0, HNCHUNK, hrow, 0)

    pltpu.sync_copy(hist_v, hist_sh.at[idx_v.at[0]], add=True)
    plsc.subcore_barrier()

    @pl.when(s < HR // 8)
    def _():
        pltpu.sync_copy(hist_sh.at[pl.ds(s * 8, 8)], hist_hbm.at[c, pl.ds(s * 8, 8)])


@functools.cache
def _hist_kernel():
    return pl.kernel(
        _hist_body,
        out_type=jax.ShapeDtypeStruct((NC, HR, D), jnp.float32),
        mesh=_mesh(),
        compiler_params=pltpu.CompilerParams(needs_layout_passes=False),
        scratch_types=[
            pltpu.VMEM((HNCHUNK, HCHUNK), jnp.int32),
            pltpu.VMEM((HR, D), jnp.float32),
            pltpu.VMEM((1, HR), jnp.int32),
            pltpu.VMEM_SHARED((HR, D), jnp.float32),
        ],
    )


def _hist_call(dst_r):
    return _hist_kernel()(dst_r)


def _prop_body(xs_hbm, src_hbm, dst_hbm, zer_hbm, out_hbm,
               src_v, dst_v, bufs, gsems, ssems, acc):
    c = lax.axis_index("c")
    s = lax.axis_index("s")
    pltpu.sync_copy(src_hbm.at[s], src_v)
    pltpu.sync_copy(dst_hbm.at[s], dst_v)
    pltpu.sync_copy(zer_hbm.at[pl.ds(s * ROWS_PT, ROWS_PT)],
                    acc.at[pl.ds(s * ROWS_PT, ROWS_PT)])
    plsc.subcore_barrier()

    xs_c = xs_hbm.at[c]

    def gather(j, t):
        pltpu.async_copy(xs_c.at[src_v.at[j]], bufs[t], gsems[t])

    def gather_wait(j, t):
        pltpu.make_async_copy(xs_c.at[src_v.at[j]], bufs[t], gsems[t]).wait()

    def scatter(j, t):
        pltpu.async_copy(bufs[t], acc.at[dst_v.at[j]], ssems[t], add=True)

    def scatter_wait(j, t):
        pltpu.make_async_copy(bufs[t], acc.at[dst_v.at[j]], ssems[t]).wait()

    gather(0, 0)
    gather(1, 1)

    def quad(q, carry):
        for t in range(NBUF):
            j = q * NBUF + t
            tp = (t + 2) % NBUF
            @pl.when(j + 2 < NCHUNK)
            def _():
                if t >= 2:
                    scatter_wait(j - 2, tp)
                else:
                    @pl.when(j >= 2)
                    def _():
                        scatter_wait(j - 2, tp)
                gather(j + 2, tp)
            gather_wait(j, t)
            scatter(j, t)
        return carry

    lax.fori_loop(0, NCHUNK // NBUF, quad, 0)
    for t in range(NBUF):
        scatter_wait(NCHUNK - NBUF + t, t)

    plsc.subcore_barrier()
    pltpu.sync_copy(acc.at[pl.ds(s * ROWS_PT, ROWS_PT)],
                    out_hbm.at[c, pl.ds(s * ROWS_PT, ROWS_PT)])


@functools.cache
def _prop_kernel():
    return pl.kernel(
        _prop_body,
        out_type=jax.ShapeDtypeStruct((NC, NPAD, HD), jnp.float32),
        mesh=_mesh(),
        compiler_params=pltpu.CompilerParams(
            needs_layout_passes=False, use_tc_tiling_on_sc=False),
        scratch_types=[
            pltpu.VMEM((NCHUNK, CHUNK), jnp.int32),
            pltpu.VMEM((NCHUNK, CHUNK), jnp.int32),
            [pltpu.VMEM((CHUNK, HD), jnp.float32) for _ in range(NBUF)],
            [pltpu.SemaphoreType.DMA for _ in range(NBUF)],
            [pltpu.SemaphoreType.DMA for _ in range(NBUF)],
            pltpu.VMEM_SHARED((NPAD, HD), jnp.float32),
        ],
    )


def _prop_call(xs2, src_r, dst_r, zer):
    return _prop_kernel()(xs2, src_r, dst_r, zer)


RBLK = 1024
NB = NPAD // RBLK


def _scale_body(h0_ref, h1_ref, x_ref, dinv_ref, xs_ref):
    deg = h0_ref[...] + h1_ref[...] + 1.0
    dinv = lax.rsqrt(deg)
    dinv_ref[...] = dinv
    xsv = x_ref[...] * dinv
    xs_ref[0] = xsv[:, :HD]
    xs_ref[1] = xsv[:, HD:]


def _scale_call(h0, h1, xp):
    return pl.pallas_call(
        _scale_body,
        grid=(NB,),
        in_specs=[
            pl.BlockSpec((RBLK, 1), lambda i: (i, 0)),
            pl.BlockSpec((RBLK, 1), lambda i: (i, 0)),
            pl.BlockSpec((RBLK, D), lambda i: (i, 0)),
        ],
        out_specs=[
            pl.BlockSpec((RBLK, 1), lambda i: (i, 0)),
            pl.BlockSpec((NC, RBLK, HD), lambda i: (0, i, 0)),
        ],
        out_shape=[
            jax.ShapeDtypeStruct((NPAD, 1), jnp.float32),
            jax.ShapeDtypeStruct((NC, NPAD, HD), jnp.float32),
        ],
    )(h0, h1, xp)


def _combine_body(x_ref, xs_ref, o_ref, dinv_ref, xn_ref, xsn_ref):
    dinv = dinv_ref[...]
    xs_full = jnp.concatenate([xs_ref[0], xs_ref[1]], axis=1)
    o_full = jnp.concatenate([o_ref[0], o_ref[1]], axis=1)
    out = dinv * (o_full + xs_full)
    xn = (1.0 - DELTA) * x_ref[...] + DELTA * out
    xn_ref[...] = xn
    xsn = dinv * xn
    xsn_ref[0] = xsn[:, :HD]
    xsn_ref[1] = xsn[:, HD:]


def _combine_call(xp, xs2, o, dinv):
    return pl.pallas_call(
        _combine_body,
        grid=(NB,),
        in_specs=[
            pl.BlockSpec((RBLK, D), lambda i: (i, 0)),
            pl.BlockSpec((NC, RBLK, HD), lambda i: (0, i, 0)),
            pl.BlockSpec((NC, RBLK, HD), lambda i: (0, i, 0)),
            pl.BlockSpec((RBLK, 1), lambda i: (i, 0)),
        ],
        out_specs=[
            pl.BlockSpec((RBLK, D), lambda i: (i, 0)),
            pl.BlockSpec((NC, RBLK, HD), lambda i: (0, i, 0)),
        ],
        out_shape=[
            jax.ShapeDtypeStruct((NPAD, D), jnp.float32),
            jax.ShapeDtypeStruct((NC, NPAD, HD), jnp.float32),
        ],
    )(xp, xs2, o, dinv)


def _matmul_body(x_ref, w_ref, b_ref, y_ref):
    y_ref[...] = jnp.dot(x_ref[...], w_ref[...],
                         preferred_element_type=jnp.float32) + b_ref[...]


def _matmul_call(xp, W, b2):
    return pl.pallas_call(
        _matmul_body,
        grid=(NB,),
        in_specs=[
            pl.BlockSpec((RBLK, D), lambda i: (i, 0)),
            pl.BlockSpec((D, D), lambda i: (0, 0)),
            pl.BlockSpec((1, D), lambda i: (0, 0)),
        ],
        out_specs=pl.BlockSpec((RBLK, D), lambda i: (i, 0)),
        out_shape=jax.ShapeDtypeStruct((NPAD, D), jnp.float32),
    )(xp, W, b2)


def kernel(x, edge_index, W, b):
    src = edge_index[0].astype(jnp.int32)
    dst = edge_index[1].astype(jnp.int32)
    src_r = jnp.pad(src, (0, EPAD - E)).reshape(NS, NCHUNK, CHUNK)
    pad_dst = N + jnp.arange(EPAD - E, dtype=jnp.int32) % (NPAD - N)
    dst_r = jnp.concatenate([dst, pad_dst]).reshape(NS, NCHUNK, CHUNK)
    dst_r32 = dst.reshape(NW, HNCHUNK, HCHUNK)
    xp = jnp.pad(x, ((0, NPAD - N), (0, 0)))
    zer = jnp.zeros((NPAD, HD), jnp.float32)

    hist = _hist_call(dst_r32)
    hflat = hist.reshape(NC, NPAD, 1)
    dinv, xs2 = _scale_call(hflat[0], hflat[1], xp)

    def step(carry, _):
        xc, xs2c = carry
        o = _prop_call(xs2c, src_r, dst_r, zer)
        xn, xsn2 = _combine_call(xc, xs2c, o, dinv)
        return (xn, xsn2), None

    (x2, _), _ = lax.scan(step, (xp, xs2), None, length=2)
    y = _matmul_call(x2, W, b.reshape(1, D))
    return y[:N]


# speedup vs baseline: 1.3453x; 1.3453x over previous
import functools

import jax
import jax.numpy as jnp
from jax import lax
from jax.experimental import pallas as pl
from jax.experimental.pallas import tpu as pltpu
from jax.experimental.pallas import tpu_sc as plsc

N = 10000
NPAD = 10240
E = 320000
D = 128
HD = D // 2
DELTA = 5.27 / 2.0

NC = 2
NS = 16
NW = NC * NS
HCHUNK = 80
HEPT = E // NW
HNCHUNK = HEPT // HCHUNK

CHUNK = 80
EPT = E // NS
NCHUNK = EPT // CHUNK
ROWS_PT = NPAD // NS
HR = NPAD // D


def _mesh():
    return plsc.VectorSubcoreMesh(
        core_axis_name="c", subcore_axis_name="s", num_cores=NC, num_subcores=NS)


def _hist_body(dst_hbm, hist_hbm, dst_v, hist_v, idx_v, hist_sh):
    c = lax.axis_index("c")
    s = lax.axis_index("s")
    wid = c * NS + s
    pltpu.sync_copy(dst_hbm.at[wid], dst_v)

    zeros16 = jnp.zeros((16,), jnp.float32)

    def zrow(r, carry):
        for i in range(D // 16):
            hist_v[r, pl.ds(i * 16, 16)] = zeros16
        return carry

    lax.fori_loop(0, HR, zrow, 0)
    for i in range(HR // 16):
        idx_v[0, pl.ds(i * 16, 16)] = lax.iota(jnp.int32, 16) + (i * 16)

    @pl.when(s == 0)
    def _():
        pltpu.sync_copy(hist_v, hist_sh)

    plsc.subcore_barrier()

    ones16 = jnp.ones((16,), jnp.float32)

    def hrow(r, carry):
        for i in range(HCHUNK // 16):
            idx = dst_v[r, pl.ds(i * 16, 16)]
            row = lax.shift_right_logical(idx, 7)
            col = lax.bitwise_and(idx, 127)
            plsc.addupdate_scatter(hist_v, [row, col], ones16)
        return carry

    lax.fori_loop(0, HNCHUNK, hrow, 0)

    pltpu.sync_copy(hist_v, hist_sh.at[idx_v.at[0]], add=True)
    plsc.subcore_barrier()

    @pl.when(s < HR // 8)
    def _():
        pltpu.sync_copy(hist_sh.at[pl.ds(s * 8, 8)], hist_hbm.at[c, pl.ds(s * 8, 8)])


@functools.cache
def _hist_kernel():
    return pl.kernel(
        _hist_body,
        out_type=jax.ShapeDtypeStruct((NC, HR, D), jnp.float32),
        mesh=_mesh(),
        compiler_params=pltpu.CompilerParams(needs_layout_passes=False),
        scratch_types=[
            pltpu.VMEM((HNCHUNK, HCHUNK), jnp.int32),
            pltpu.VMEM((HR, D), jnp.float32),
            pltpu.VMEM((1, HR), jnp.int32),
            pltpu.VMEM_SHARED((HR, D), jnp.float32),
        ],
    )


def _hist_call(dst_r):
    return _hist_kernel()(dst_r)


def _prop_body(xs_hbm, src_hbm, dst_hbm, zer_hbm, out_hbm,
               src_v, dst_v, bufs, gsems, ssems, acc):
    c = lax.axis_index("c")
    s = lax.axis_index("s")
    pltpu.sync_copy(src_hbm.at[s], src_v)
    pltpu.sync_copy(dst_hbm.at[s], dst_v)
    pltpu.sync_copy(zer_hbm.at[pl.ds(s * ROWS_PT, ROWS_PT)],
                    acc.at[pl.ds(s * ROWS_PT, ROWS_PT)])
    plsc.subcore_barrier()

    xs_c = xs_hbm.at[c]

    pltpu.async_copy(xs_c.at[src_v.at[0]], bufs[0], gsems[0])

    def step2(p, carry):
        j = p * 2
        pltpu.make_async_copy(xs_c.at[src_v.at[j]], bufs[0], gsems[0]).wait()
        pltpu.async_copy(xs_c.at[src_v.at[j + 1]], bufs[1], gsems[1])
        pltpu.sync_copy(bufs[0], acc.at[dst_v.at[j]], add=True)
        pltpu.make_async_copy(xs_c.at[src_v.at[j + 1]], bufs[1], gsems[1]).wait()

        @pl.when(j + 2 < NCHUNK)
        def _():
            pltpu.async_copy(xs_c.at[src_v.at[j + 2]], bufs[0], gsems[0])

        pltpu.sync_copy(bufs[1], acc.at[dst_v.at[j + 1]], add=True)
        return carry

    lax.fori_loop(0, NCHUNK // 2, step2, 0)

    plsc.subcore_barrier()
    pltpu.sync_copy(acc.at[pl.ds(s * ROWS_PT, ROWS_PT)],
                    out_hbm.at[c, pl.ds(s * ROWS_PT, ROWS_PT)])


@functools.cache
def _prop_kernel():
    return pl.kernel(
        _prop_body,
        out_type=jax.ShapeDtypeStruct((NC, NPAD, HD), jnp.float32),
        mesh=_mesh(),
        compiler_params=pltpu.CompilerParams(
            needs_layout_passes=False, use_tc_tiling_on_sc=False),
        scratch_types=[
            pltpu.VMEM((NCHUNK, CHUNK), jnp.int32),
            pltpu.VMEM((NCHUNK, CHUNK), jnp.int32),
            [pltpu.VMEM((CHUNK, HD), jnp.float32) for _ in range(2)],
            [pltpu.SemaphoreType.DMA for _ in range(2)],
            [pltpu.SemaphoreType.DMA for _ in range(2)],
            pltpu.VMEM_SHARED((NPAD, HD), jnp.float32),
        ],
    )


def _prop_call(xs2, src_r, dst_r, zer):
    return _prop_kernel()(xs2, src_r, dst_r, zer)


RBLK = 1024
NB = NPAD // RBLK


def _scale_body(h0_ref, h1_ref, x_ref, dinv_ref, xs_ref):
    deg = h0_ref[...] + h1_ref[...] + 1.0
    dinv = lax.rsqrt(deg)
    dinv_ref[...] = dinv
    xsv = x_ref[...] * dinv
    xs_ref[0] = xsv[:, :HD]
    xs_ref[1] = xsv[:, HD:]


def _scale_call(h0, h1, xp):
    return pl.pallas_call(
        _scale_body,
        grid=(NB,),
        in_specs=[
            pl.BlockSpec((RBLK, 1), lambda i: (i, 0)),
            pl.BlockSpec((RBLK, 1), lambda i: (i, 0)),
            pl.BlockSpec((RBLK, D), lambda i: (i, 0)),
        ],
        out_specs=[
            pl.BlockSpec((RBLK, 1), lambda i: (i, 0)),
            pl.BlockSpec((NC, RBLK, HD), lambda i: (0, i, 0)),
        ],
        out_shape=[
            jax.ShapeDtypeStruct((NPAD, 1), jnp.float32),
            jax.ShapeDtypeStruct((NC, NPAD, HD), jnp.float32),
        ],
    )(h0, h1, xp)


def _combine_body(x_ref, xs_ref, o_ref, dinv_ref, xn_ref, xsn_ref):
    dinv = dinv_ref[...]
    xs_full = jnp.concatenate([xs_ref[0], xs_ref[1]], axis=1)
    o_full = jnp.concatenate([o_ref[0], o_ref[1]], axis=1)
    out = dinv * (o_full + xs_full)
    xn = (1.0 - DELTA) * x_ref[...] + DELTA * out
    xn_ref[...] = xn
    xsn = dinv * xn
    xsn_ref[0] = xsn[:, :HD]
    xsn_ref[1] = xsn[:, HD:]


def _combine_call(xp, xs2, o, dinv):
    return pl.pallas_call(
        _combine_body,
        grid=(NB,),
        in_specs=[
            pl.BlockSpec((RBLK, D), lambda i: (i, 0)),
            pl.BlockSpec((NC, RBLK, HD), lambda i: (0, i, 0)),
            pl.BlockSpec((NC, RBLK, HD), lambda i: (0, i, 0)),
            pl.BlockSpec((RBLK, 1), lambda i: (i, 0)),
        ],
        out_specs=[
            pl.BlockSpec((RBLK, D), lambda i: (i, 0)),
            pl.BlockSpec((NC, RBLK, HD), lambda i: (0, i, 0)),
        ],
        out_shape=[
            jax.ShapeDtypeStruct((NPAD, D), jnp.float32),
            jax.ShapeDtypeStruct((NC, NPAD, HD), jnp.float32),
        ],
    )(xp, xs2, o, dinv)


def _matmul_body(x_ref, w_ref, b_ref, y_ref):
    y_ref[...] = jnp.dot(x_ref[...], w_ref[...],
                         preferred_element_type=jnp.float32) + b_ref[...]


def _matmul_call(xp, W, b2):
    return pl.pallas_call(
        _matmul_body,
        grid=(NB,),
        in_specs=[
            pl.BlockSpec((RBLK, D), lambda i: (i, 0)),
            pl.BlockSpec((D, D), lambda i: (0, 0)),
            pl.BlockSpec((1, D), lambda i: (0, 0)),
        ],
        out_specs=pl.BlockSpec((RBLK, D), lambda i: (i, 0)),
        out_shape=jax.ShapeDtypeStruct((NPAD, D), jnp.float32),
    )(xp, W, b2)


def kernel(x, edge_index, W, b):
    src = edge_index[0].astype(jnp.int32)
    dst = edge_index[1].astype(jnp.int32)
    src_r = src.reshape(NS, NCHUNK, CHUNK)
    dst_r = dst.reshape(NS, NCHUNK, CHUNK)
    dst_r32 = dst.reshape(NW, HNCHUNK, HCHUNK)
    xp = jnp.pad(x, ((0, NPAD - N), (0, 0)))
    zer = jnp.zeros((NPAD, HD), jnp.float32)

    hist = _hist_call(dst_r32)
    hflat = hist.reshape(NC, NPAD, 1)
    dinv, xs2 = _scale_call(hflat[0], hflat[1], xp)

    def step(carry, _):
        xc, xs2c = carry
        o = _prop_call(xs2c, src_r, dst_r, zer)
        xn, xsn2 = _combine_call(xc, xs2c, o, dinv)
        return (xn, xsn2), None

    (x2, _), _ = lax.scan(step, (xp, xs2), None, length=2)
    y = _matmul_call(x2, W, b.reshape(1, D))
    return y[:N]
